# Initial kernel scaffold; baseline (speedup 1.0000x reference)
import jax
import jax.numpy as jnp
from jax.experimental import pallas as pl


def kernel(x, input_table):
    raise NotImplementedError("write your pallas kernel here")


# speedup vs baseline: 5.2472x; 5.2472x over previous
import functools

import jax
import jax.numpy as jnp
from jax import lax
from jax.experimental import pallas as pl
from jax.experimental.pallas import tpu as pltpu
from jax.experimental.pallas import tpu_sc as plsc

B = 4096
L = 200
E = 128
V = 1000

NC = 2
NS = 16
NW = NC * NS

ROWS = B * L
RPW = ROWS // NW
G = L
NG = RPW // G

_mesh = plsc.VectorSubcoreMesh(core_axis_name="c", subcore_axis_name="s")


@functools.partial(
    pl.kernel,
    out_type=jax.ShapeDtypeStruct((ROWS, E), jnp.float32),
    mesh=_mesh,
    scratch_types=[
        pltpu.VMEM((RPW,), jnp.int32),
        pltpu.VMEM((G, E), jnp.float32),
        pltpu.SemaphoreType.DMA,
    ],
)
def _emb_kernel(table_hbm, xflat_hbm, out_hbm, idx_v, buf, sem):
    wid = lax.axis_index("s") * NC + lax.axis_index("c")
    base = wid * RPW
    pltpu.sync_copy(xflat_hbm.at[pl.ds(base, RPW)], idx_v)

    def group(g, carry):
        cp1 = pltpu.async_copy(
            table_hbm.at[idx_v.at[pl.ds(g * G, 128)]],
            buf.at[pl.ds(0, 128)],
            sem,
        )
        cp2 = pltpu.async_copy(
            table_hbm.at[idx_v.at[pl.ds(g * G + 128, G - 128)]],
            buf.at[pl.ds(128, G - 128)],
            sem,
        )
        cp1.wait()
        cp2.wait()

        def addrow(j, carry2):
            jf = j.astype(jnp.float32)
            val = jnp.where(j % 2 == 0, jf * jnp.float32(1e-8),
                            jnp.float32(1.0))
            for e in range(E // 16):
                buf[j, pl.ds(e * 16, 16)] = buf[j, pl.ds(e * 16, 16)] + val
            return carry2

        lax.fori_loop(0, G, addrow, 0, unroll=2)

        pltpu.sync_copy(buf, out_hbm.at[pl.ds(base + g * G, G)])
        return carry

    lax.fori_loop(0, NG, group, 0)


def kernel(x, input_table):
    x_flat = x.reshape(ROWS).astype(jnp.int32)
    out = _emb_kernel(input_table, x_flat)
    return out.reshape(B, L, E)


# speedup vs baseline: 6.5252x; 1.2435x over previous
import functools

import jax
import jax.numpy as jnp
from jax import lax
from jax.experimental import pallas as pl
from jax.experimental.pallas import tpu as pltpu
from jax.experimental.pallas import tpu_sc as plsc

B = 4096
L = 200
E = 128
V = 1000

NC = 2
NS = 16
NW = NC * NS

ROWS = B * L
RPW = ROWS // NW
G = L
NG = RPW // G
NBUF = 4
AHEAD = 2

_mesh = plsc.VectorSubcoreMesh(core_axis_name="c", subcore_axis_name="s")


@functools.partial(
    pl.kernel,
    out_type=jax.ShapeDtypeStruct((ROWS, E), jnp.float32),
    mesh=_mesh,
    scratch_types=[
        pltpu.VMEM((RPW,), jnp.int32),
        pltpu.VMEM((NBUF, G, E), jnp.float32),
        pltpu.SemaphoreType.DMA((NBUF,)),
        pltpu.SemaphoreType.DMA((NBUF,)),
    ],
)
def _emb_kernel(table_hbm, xflat_hbm, out_hbm, idx_v, bufs, gsem, wsem):
    wid = lax.axis_index("s") * NC + lax.axis_index("c")
    base = wid * RPW
    pltpu.sync_copy(xflat_hbm.at[pl.ds(base, RPW)], idx_v)

    def gather_descs(g, slot):
        d1 = pltpu.make_async_copy(
            table_hbm.at[idx_v.at[pl.ds(g * G, 128)]],
            bufs.at[slot, pl.ds(0, 128)],
            gsem.at[slot],
        )
        d2 = pltpu.make_async_copy(
            table_hbm.at[idx_v.at[pl.ds(g * G + 128, G - 128)]],
            bufs.at[slot, pl.ds(128, G - 128)],
            gsem.at[slot],
        )
        return d1, d2

    def write_desc(g, slot):
        return pltpu.make_async_copy(
            bufs.at[slot],
            out_hbm.at[pl.ds(base + g * G, G)],
            wsem.at[slot],
        )

    for g in range(AHEAD):
        d1, d2 = gather_descs(g, g % NBUF)
        d1.start()
        d2.start()

    def group(g, carry):
        slot = lax.rem(g, NBUF)

        @pl.when(g + AHEAD < NG)
        def _():
            s2 = lax.rem(g + AHEAD, NBUF)

            @pl.when(g >= NBUF - AHEAD)
            def _():
                write_desc(g - (NBUF - AHEAD), s2).wait()

            d1, d2 = gather_descs(g + AHEAD, s2)
            d1.start()
            d2.start()

        d1, d2 = gather_descs(g, slot)
        d1.wait()
        d2.wait()

        def addrow(j, carry2):
            jf = j.astype(jnp.float32)
            val = jnp.where(j % 2 == 0, jf * jnp.float32(1e-8),
                            jnp.float32(1.0))
            for e in range(E // 16):
                bufs[slot, j, pl.ds(e * 16, 16)] = (
                    bufs[slot, j, pl.ds(e * 16, 16)] + val)
            return carry2

        lax.fori_loop(0, G, addrow, 0, unroll=2)

        write_desc(g, slot).start()
        return carry

    lax.fori_loop(0, NG, group, 0)

    for g in range(NG - NBUF, NG):
        write_desc(g, g % NBUF).wait()


def kernel(x, input_table):
    x_flat = x.reshape(ROWS).astype(jnp.int32)
    out = _emb_kernel(input_table, x_flat)
    return out.reshape(B, L, E)
